# Initial kernel scaffold; baseline (speedup 1.0000x reference)
import jax
import jax.numpy as jnp
from jax.experimental import pallas as pl


def kernel(logits, targets):
    raise NotImplementedError("write your pallas kernel here")


# speedup vs baseline: 1.3663x; 1.3663x over previous
import jax
import jax.numpy as jnp
import numpy as np
from jax.experimental import pallas as pl
from jax.experimental.pallas import tpu as pltpu

_CLIP = 0.05
_ALPHA_UNANN = 0.5
_K_PER_ROW = 5
_INT_MIN = np.int32(-2147483648)
_MASK31 = np.int32(2147483647)


def _pass1_body(logits_ref, targets_ref, sum_ref, keys_ref):
    i = pl.program_id(0)
    l = logits_ref[...]
    t = targets_ref[...]
    p = jax.nn.sigmoid(l)
    xn = jnp.minimum((1.0 - p) + _CLIP, 1.0)
    one_m_xn = 1.0 - xn
    log_p = jnp.log(jnp.maximum(p, 1e-8))
    log_xn = jnp.log(jnp.maximum(xn, 1e-8))
    pos = t == 1
    una = t == -1
    sq = one_m_xn * one_m_xn
    contrib = jnp.where(
        pos, log_p,
        jnp.where(una, _ALPHA_UNANN * log_xn * sq, log_xn * (sq * sq)))
    partial = jnp.sum(contrib)
    prev = jnp.where(i == 0, 0.0, sum_ref[0, 0])
    sum_ref[0, 0] = prev + partial

    bits = jax.lax.bitcast_convert_type(l, jnp.int32)
    key = jnp.where(bits >= 0, bits, bits ^ _MASK31)
    keys_ref[...] = jnp.where(una, key, _INT_MIN)


def _unkey_f(keys):
    lb = jnp.where(keys >= 0, keys, keys ^ _MASK31)
    lv = jax.lax.bitcast_convert_type(lb, jnp.float32)
    p = jax.nn.sigmoid(lv)
    xn = jnp.minimum((1.0 - p) + _CLIP, 1.0)
    f = _ALPHA_UNANN * jnp.log(jnp.maximum(xn, 1e-8)) * (1.0 - xn) ** 2
    return jnp.where(keys == _INT_MIN, 0.0, f)


def kernel(logits, targets):
    B, C = logits.shape
    RB = 8
    grid = B // RB
    s_all, keys = pl.pallas_call(
        _pass1_body,
        grid=(grid,),
        in_specs=[
            pl.BlockSpec((RB, C), lambda i: (i, 0)),
            pl.BlockSpec((RB, C), lambda i: (i, 0)),
        ],
        out_specs=[
            pl.BlockSpec((1, 1), lambda i: (0, 0), memory_space=pltpu.SMEM),
            pl.BlockSpec((RB, C), lambda i: (i, 0)),
        ],
        out_shape=[
            jax.ShapeDtypeStruct((1, 1), jnp.float32),
            jax.ShapeDtypeStruct((B, C), jnp.int32),
        ],
    )(logits, targets)

    k = _K_PER_ROW * B
    topvals, _ = jax.lax.top_k(keys.reshape(-1), k)
    d_sum = jnp.sum(_unkey_f(topvals))
    return d_sum - s_all[0, 0]


# speedup vs baseline: 30.7282x; 22.4894x over previous
import functools

import jax
import jax.numpy as jnp
import numpy as np
from jax import lax
from jax.experimental import pallas as pl
from jax.experimental.pallas import tpu as pltpu
from jax.experimental.pallas import tpu_sc as plsc

_CLIP = 0.05
_ALPHA_UNANN = 0.5
_K_PER_ROW = 5
_INT_MIN = np.int32(-2147483648)
_MASK31 = np.int32(2147483647)

_NW = 32
_NBINS12 = 4096
_NBINS3 = 256


def _plan(B, C):
    cpad = ((C + 127) // 128) * 128
    rows_w = B // _NW
    cc = None
    for d in range(3200 - 3200 % 128, 127, -128):
        if cpad % d == 0:
            cc = d
            break
    n_chunks = (rows_w // 8) * (cpad // cc)
    assert n_chunks % 2 == 0 and rows_w % 8 == 0
    return cpad, rows_w, cc, n_chunks


def _pass1_body(cpad, logits_ref, targets_ref, sum_ref, keys_ref):
    i = pl.program_id(0)
    l = logits_ref[...]
    t = targets_ref[...]
    p = jax.nn.sigmoid(l)
    xn = jnp.minimum((1.0 - p) + _CLIP, 1.0)
    one_m_xn = 1.0 - xn
    log_p = jnp.log(jnp.maximum(p, 1e-8))
    log_xn = jnp.log(jnp.maximum(xn, 1e-8))
    pos = t == 1
    una = t == -1
    sq = one_m_xn * one_m_xn
    contrib = jnp.where(
        pos, log_p,
        jnp.where(una, _ALPHA_UNANN * log_xn * sq, log_xn * (sq * sq)))
    partial = jnp.sum(contrib)
    prev = jnp.where(i == 0, 0.0, sum_ref[0, 0])
    sum_ref[0, 0] = prev + partial

    bits = jax.lax.bitcast_convert_type(l, jnp.int32)
    key = jnp.where(bits >= 0, bits, bits ^ _MASK31)
    key = jnp.where(una, key, _INT_MIN)
    rb, c = key.shape
    if cpad > c:
        pad = jnp.full((rb, cpad - c), _INT_MIN, jnp.int32)
        key = jnp.concatenate([key, pad], axis=1)
    keys_ref[...] = key


def _unkey_f(keys):
    lb = jnp.where(keys >= 0, keys, keys ^ _MASK31)
    lv = jax.lax.bitcast_convert_type(lb, jnp.float32)
    p = jax.nn.sigmoid(lv)
    xn = jnp.minimum((1.0 - p) + _CLIP, 1.0)
    f = _ALPHA_UNANN * jnp.log(jnp.maximum(xn, 1e-8)) * (1.0 - xn) ** 2
    return jnp.where(keys == _INT_MIN, 0.0, f)


def _make_hist(B, cpad, rows_w, cc, n_chunks, nbins, shift, pref_shift):
    mesh = plsc.VectorSubcoreMesh(core_axis_name="c", subcore_axis_name="s",
                                  num_cores=2, num_subcores=16)
    cpr = cpad // cc
    nv = cc // 16
    has_prefix = pref_shift is not None

    scratch = [
        pltpu.VMEM((16 * nbins,), jnp.int32),
        pltpu.VMEM((8, cc), jnp.int32),
        pltpu.VMEM((8, cc), jnp.int32),
        pltpu.VMEM((nbins,), jnp.int32),
    ]
    if has_prefix:
        scratch.append(pltpu.VMEM((16,), jnp.int32))
    scratch += [pltpu.SemaphoreType.DMA, pltpu.SemaphoreType.DMA]

    def body(*args):
        if has_prefix:
            (keys_hbm, pref_hbm, out_hbm,
             hist, buf0, buf1, merged, pref_v, sem0, sem1) = args
        else:
            (keys_hbm, out_hbm,
             hist, buf0, buf1, merged, sem0, sem1) = args

        cid = lax.axis_index("c")
        sid = lax.axis_index("s")
        w = sid * 2 + cid
        row0 = w * rows_w

        zeros16 = jnp.zeros((16,), jnp.int32)

        def zb(j, _):
            hist[pl.ds(j * 16, 16)] = zeros16
            return 0
        lax.fori_loop(0, nbins, zb, 0)

        if has_prefix:
            pltpu.sync_copy(pref_hbm, pref_v)
            pref = pref_v[...]

        ones = jnp.ones((16,), jnp.int32)
        lane_base = lax.iota(jnp.int32, 16) * nbins
        shift_v = jnp.full((16,), shift, jnp.int32)
        if has_prefix:
            pshift_v = jnp.full((16,), pref_shift, jnp.int32)

        def start(f, buf, sem):
            r = row0 + 8 * (f // cpr)
            c = (f % cpr) * cc
            pltpu.make_async_copy(
                keys_hbm.at[pl.ds(r, 8), pl.ds(c, cc)], buf, sem).start()

        def wait(buf, sem):
            pltpu.make_async_copy(
                keys_hbm.at[pl.ds(row0, 8), pl.ds(0, cc)], buf, sem).wait()

        def process(buf):
            for r in range(8):
                def pb(j, _, r=r):
                    kv = buf[r, pl.ds(j * 16, 16)]
                    u = kv ^ _INT_MIN
                    fld = lax.shift_right_logical(u, shift_v)
                    if has_prefix:
                        okm = lax.shift_right_logical(u, pshift_v) == pref
                        fld = fld & (nbins - 1)
                        plsc.addupdate_scatter(hist, [lane_base + fld], ones,
                                               mask=okm)
                    else:
                        plsc.addupdate_scatter(hist, [lane_base + fld], ones)
                    return 0
                lax.fori_loop(0, nv, pb, 0)

        start(0, buf0, sem0)

        def pair(t, _):
            start(2 * t + 1, buf1, sem1)
            wait(buf0, sem0)
            process(buf0)

            @pl.when(t + 1 < n_chunks // 2)
            def _():
                start(2 * t + 2, buf0, sem0)

            wait(buf1, sem1)
            process(buf1)
            return 0

        lax.fori_loop(0, n_chunks // 2, pair, 0)

        def mb(cb, _):
            acc = hist[pl.ds(cb * 16, 16)]
            for r in range(1, 16):
                acc = acc + hist[pl.ds(r * nbins + cb * 16, 16)]
            merged[pl.ds(cb * 16, 16)] = acc
            return 0
        lax.fori_loop(0, nbins // 16, mb, 0)

        pltpu.sync_copy(merged, out_hbm.at[pl.ds(w * nbins, nbins)])

    return pl.kernel(
        body,
        out_type=jax.ShapeDtypeStruct((_NW * nbins,), jnp.int32),
        mesh=mesh,
        scratch_types=scratch,
        compiler_params=pltpu.CompilerParams(needs_layout_passes=False),
    )


def _select(hist, k_rem):
    h = jnp.sum(hist.reshape(_NW, -1), axis=0)
    c = jnp.cumsum(h[::-1])[::-1]
    jstar = jnp.sum((c >= k_rem).astype(jnp.int32)) - 1
    above = c[jstar] - h[jstar]
    return jstar, k_rem - above


def _dpass_body(t_ref, keys_ref, out_ref):
    i = pl.program_id(0)
    t = t_ref[0, 0]
    kv = keys_ref[...]
    f = _unkey_f(kv)
    d = jnp.sum(jnp.where(kv > t, f, 0.0))
    prev = jnp.where(i == 0, 0.0, out_ref[0, 0])
    out_ref[0, 0] = prev + d


@jax.jit
def kernel(logits, targets):
    B, C = logits.shape
    cpad, rows_w, chunk, n_chunks = _plan(B, C)
    RB = 8
    grid = B // RB
    k = _K_PER_ROW * B

    s_all, keys = pl.pallas_call(
        functools.partial(_pass1_body, cpad),
        grid=(grid,),
        in_specs=[
            pl.BlockSpec((RB, C), lambda i: (i, 0)),
            pl.BlockSpec((RB, C), lambda i: (i, 0)),
        ],
        out_specs=[
            pl.BlockSpec((1, 1), lambda i: (0, 0), memory_space=pltpu.SMEM),
            pl.BlockSpec((RB, cpad), lambda i: (i, 0)),
        ],
        out_shape=[
            jax.ShapeDtypeStruct((1, 1), jnp.float32),
            jax.ShapeDtypeStruct((B, cpad), jnp.int32),
        ],
    )(logits, targets)

    hist1 = _make_hist(B, cpad, rows_w, chunk, n_chunks, _NBINS12, 20, None)
    hist2 = _make_hist(B, cpad, rows_w, chunk, n_chunks, _NBINS12, 8, 20)
    hist3 = _make_hist(B, cpad, rows_w, chunk, n_chunks, _NBINS3, 0, 8)

    h1 = hist1(keys)
    j1, kr1 = _select(h1, jnp.int32(k))
    h2 = hist2(keys, jnp.full((16,), j1, jnp.int32))
    j2, kr2 = _select(h2, kr1)
    p24 = j1 * _NBINS12 + j2
    h3 = hist3(keys, jnp.full((16,), p24, jnp.int32))
    j3, m = _select(h3, kr2)

    t_u = (p24 << 8) | j3
    t_key = t_u ^ jnp.int32(-2147483648)

    d_gt = pl.pallas_call(
        _dpass_body,
        grid=(grid,),
        in_specs=[
            pl.BlockSpec((1, 1), lambda i: (0, 0), memory_space=pltpu.SMEM),
            pl.BlockSpec((RB, cpad), lambda i: (i, 0)),
        ],
        out_specs=pl.BlockSpec((1, 1), lambda i: (0, 0),
                               memory_space=pltpu.SMEM),
        out_shape=jax.ShapeDtypeStruct((1, 1), jnp.float32),
    )(t_key.reshape(1, 1), keys)

    f_t = _unkey_f(t_key.reshape(1))[0]
    d_sum = d_gt[0, 0] + m.astype(jnp.float32) * f_t
    return d_sum - s_all[0, 0]


# speedup vs baseline: 65.7379x; 2.1393x over previous
import functools

import jax
import jax.numpy as jnp
import numpy as np
from jax import lax
from jax.experimental import pallas as pl
from jax.experimental.pallas import tpu as pltpu
from jax.experimental.pallas import tpu_sc as plsc

_CLIP = 0.05
_ALPHA_UNANN = 0.5
_K_PER_ROW = 5
_INT_MIN = np.int32(-2147483648)
_MASK31 = np.int32(2147483647)

_NW = 32
_NBINS12 = 4096
_NBINS3 = 256


def _plan(B, C):
    cpad = ((C + 127) // 128) * 128
    rows_w = B // _NW
    cc = None
    for d in range(3200 - 3200 % 128, 127, -128):
        if cpad % d == 0:
            cc = d
            break
    n_chunks = (rows_w // 8) * (cpad // cc)
    assert n_chunks % 2 == 0 and rows_w % 8 == 0
    return cpad, rows_w, cc, n_chunks


def _pass1_body(cpad, logits_ref, targets_ref, sum_ref, keys_ref):
    i = pl.program_id(0)
    l = logits_ref[...]
    t = targets_ref[...]
    p = jax.nn.sigmoid(l)
    xn = jnp.minimum((1.0 - p) + _CLIP, 1.0)
    one_m_xn = 1.0 - xn
    log_p = jnp.log(jnp.maximum(p, 1e-8))
    log_xn = jnp.log(jnp.maximum(xn, 1e-8))
    pos = t == 1
    una = t == -1
    sq = one_m_xn * one_m_xn
    contrib = jnp.where(
        pos, log_p,
        jnp.where(una, _ALPHA_UNANN * log_xn * sq, log_xn * (sq * sq)))
    partial = jnp.sum(contrib)
    prev = jnp.where(i == 0, 0.0, sum_ref[0, 0])
    sum_ref[0, 0] = prev + partial

    bits = jax.lax.bitcast_convert_type(l, jnp.int32)
    key = jnp.where(bits >= 0, bits, bits ^ _MASK31)
    key = jnp.where(una, key, _INT_MIN)
    rb, c = key.shape
    if cpad > c:
        pad = jnp.full((rb, cpad - c), _INT_MIN, jnp.int32)
        key = jnp.concatenate([key, pad], axis=1)
    keys_ref[...] = key


def _unkey_f(keys):
    lb = jnp.where(keys >= 0, keys, keys ^ _MASK31)
    lv = jax.lax.bitcast_convert_type(lb, jnp.float32)
    p = jax.nn.sigmoid(lv)
    xn = jnp.minimum((1.0 - p) + _CLIP, 1.0)
    f = _ALPHA_UNANN * jnp.log(jnp.maximum(xn, 1e-8)) * (1.0 - xn) ** 2
    return jnp.where(keys == _INT_MIN, 0.0, f)


def _make_hist(B, cpad, rows_w, cc, n_chunks, nbins, shift, pref_shift):
    mesh = plsc.VectorSubcoreMesh(core_axis_name="c", subcore_axis_name="s",
                                  num_cores=2, num_subcores=16)
    cpr = cpad // cc
    nv = cc // 16
    has_prefix = pref_shift is not None

    scratch = [
        pltpu.VMEM((16 * nbins,), jnp.int32),
        pltpu.VMEM((8, cc), jnp.int32),
        pltpu.VMEM((8, cc), jnp.int32),
        pltpu.VMEM((nbins,), jnp.int32),
    ]
    if has_prefix:
        scratch.append(pltpu.VMEM((16,), jnp.int32))
    scratch += [pltpu.SemaphoreType.DMA, pltpu.SemaphoreType.DMA]

    def body(*args):
        if has_prefix:
            (keys_hbm, pref_hbm, out_hbm,
             hist, buf0, buf1, merged, pref_v, sem0, sem1) = args
        else:
            (keys_hbm, out_hbm,
             hist, buf0, buf1, merged, sem0, sem1) = args

        cid = lax.axis_index("c")
        sid = lax.axis_index("s")
        w = sid * 2 + cid
        row0 = w * rows_w

        zeros16 = jnp.zeros((16,), jnp.int32)

        def zb(j, _):
            hist[pl.ds(j * 16, 16)] = zeros16
            return 0
        lax.fori_loop(0, nbins, zb, 0)

        if has_prefix:
            pltpu.sync_copy(pref_hbm, pref_v)
            pref = pref_v[...]

        ones = jnp.ones((16,), jnp.int32)
        lane_base = lax.iota(jnp.int32, 16) * nbins
        shift_v = jnp.full((16,), shift, jnp.int32)
        if has_prefix:
            pshift_v = jnp.full((16,), pref_shift, jnp.int32)

        def start(f, buf, sem):
            r = row0 + 8 * (f // cpr)
            c = (f % cpr) * cc
            pltpu.make_async_copy(
                keys_hbm.at[pl.ds(r, 8), pl.ds(c, cc)], buf, sem).start()

        def wait(buf, sem):
            pltpu.make_async_copy(
                keys_hbm.at[pl.ds(row0, 8), pl.ds(0, cc)], buf, sem).wait()

        def process(buf):
            for r in range(8):
                @plsc.parallel_loop(0, nv, 1, unroll=8)
                def pb(j, r=r):
                    kv = buf[r, pl.ds(j * 16, 16)]
                    fld = lax.shift_right_logical(kv, shift_v)
                    if has_prefix:
                        okm = lax.shift_right_logical(kv, pshift_v) == pref
                        fld = fld & (nbins - 1)
                        plsc.addupdate_scatter(hist, [lane_base + fld], ones,
                                               mask=okm)
                    else:
                        plsc.addupdate_scatter(hist, [lane_base + fld], ones)

        start(0, buf0, sem0)

        def pair(t, _):
            start(2 * t + 1, buf1, sem1)
            wait(buf0, sem0)
            process(buf0)

            @pl.when(t + 1 < n_chunks // 2)
            def _():
                start(2 * t + 2, buf0, sem0)

            wait(buf1, sem1)
            process(buf1)
            return 0

        lax.fori_loop(0, n_chunks // 2, pair, 0)

        def mb(cb, _):
            acc = hist[pl.ds(cb * 16, 16)]
            for r in range(1, 16):
                acc = acc + hist[pl.ds(r * nbins + cb * 16, 16)]
            merged[pl.ds(cb * 16, 16)] = acc
            return 0
        lax.fori_loop(0, nbins // 16, mb, 0)

        pltpu.sync_copy(merged, out_hbm.at[pl.ds(w * nbins, nbins)])

    return pl.kernel(
        body,
        out_type=jax.ShapeDtypeStruct((_NW * nbins,), jnp.int32),
        mesh=mesh,
        scratch_types=scratch,
        compiler_params=pltpu.CompilerParams(needs_layout_passes=False),
    )


def _select(h, k_rem):
    c = jnp.cumsum(h[::-1])[::-1]
    jstar = jnp.sum((c >= k_rem).astype(jnp.int32)) - 1
    above = c[jstar] - h[jstar]
    return jstar, k_rem - above


def _radix_threshold(keys, B, cpad, rows_w, cc, n_chunks, k):
    hist1 = _make_hist(B, cpad, rows_w, cc, n_chunks, _NBINS12, 20, None)
    hist2 = _make_hist(B, cpad, rows_w, cc, n_chunks, _NBINS12, 8, 20)
    hist3 = _make_hist(B, cpad, rows_w, cc, n_chunks, _NBINS3, 0, 8)

    h1 = jnp.sum(hist1(keys).reshape(_NW, _NBINS12), axis=0)
    h1u = h1.reshape(2, _NBINS12 // 2)[::-1].reshape(-1)
    j1u, kr1 = _select(h1u, jnp.int32(k))
    praw1 = j1u ^ (_NBINS12 // 2)
    h2 = jnp.sum(hist2(keys, jnp.full((16,), praw1, jnp.int32))
                 .reshape(_NW, _NBINS12), axis=0)
    j2, kr2 = _select(h2, kr1)
    praw24 = praw1 * _NBINS12 + j2
    h3 = jnp.sum(hist3(keys, jnp.full((16,), praw24, jnp.int32))
                 .reshape(_NW, _NBINS3), axis=0)
    j3, m = _select(h3, kr2)

    t_key = (praw24 << 8) | j3
    return t_key, m


def _dpass_body(t_ref, keys_ref, out_ref):
    i = pl.program_id(0)
    t = t_ref[0, 0]
    kv = keys_ref[...]
    f = _unkey_f(kv)
    d = jnp.sum(jnp.where(kv > t, f, 0.0))
    prev = jnp.where(i == 0, 0.0, out_ref[0, 0])
    out_ref[0, 0] = prev + d


@jax.jit
def kernel(logits, targets):
    B, C = logits.shape
    cpad, rows_w, chunk, n_chunks = _plan(B, C)
    RB = 8
    grid = B // RB
    k = _K_PER_ROW * B

    s_all, keys = pl.pallas_call(
        functools.partial(_pass1_body, cpad),
        grid=(grid,),
        in_specs=[
            pl.BlockSpec((RB, C), lambda i: (i, 0)),
            pl.BlockSpec((RB, C), lambda i: (i, 0)),
        ],
        out_specs=[
            pl.BlockSpec((1, 1), lambda i: (0, 0), memory_space=pltpu.SMEM),
            pl.BlockSpec((RB, cpad), lambda i: (i, 0)),
        ],
        out_shape=[
            jax.ShapeDtypeStruct((1, 1), jnp.float32),
            jax.ShapeDtypeStruct((B, cpad), jnp.int32),
        ],
    )(logits, targets)

    t_key, m = _radix_threshold(keys, B, cpad, rows_w, chunk, n_chunks, k)

    d_gt = pl.pallas_call(
        _dpass_body,
        grid=(grid,),
        in_specs=[
            pl.BlockSpec((1, 1), lambda i: (0, 0), memory_space=pltpu.SMEM),
            pl.BlockSpec((RB, cpad), lambda i: (i, 0)),
        ],
        out_specs=pl.BlockSpec((1, 1), lambda i: (0, 0),
                               memory_space=pltpu.SMEM),
        out_shape=jax.ShapeDtypeStruct((1, 1), jnp.float32),
    )(t_key.reshape(1, 1), keys)

    f_t = _unkey_f(t_key.reshape(1))[0]
    d_sum = d_gt[0, 0] + m.astype(jnp.float32) * f_t
    return d_sum - s_all[0, 0]


# speedup vs baseline: 90.4603x; 1.3761x over previous
import functools

import jax
import jax.numpy as jnp
import numpy as np
from jax import lax
from jax.experimental import pallas as pl
from jax.experimental.pallas import tpu as pltpu
from jax.experimental.pallas import tpu_sc as plsc

_CLIP = 0.05
_ALPHA_UNANN = 0.5
_K_PER_ROW = 5
_INT_MIN = np.int32(-2147483648)
_MASK31 = np.int32(2147483647)

_NW = 32
_NBINS12 = 4096
_NBINS3 = 256


def _plan(B, C):
    cpad = ((C + 127) // 128) * 128
    rows_w = B // _NW
    cc = None
    for d in range(3200 - 3200 % 128, 127, -128):
        if cpad % d == 0:
            cc = d
            break
    n_chunks = (rows_w // 8) * (cpad // cc)
    assert n_chunks % 2 == 0 and rows_w % 8 == 0
    return cpad, rows_w, cc, n_chunks


def _pass1_body(cpad, logits_ref, targets_ref, sum_ref, keys_ref):
    i = pl.program_id(0)
    l = logits_ref[...]
    t = targets_ref[...]
    p = jax.nn.sigmoid(l)
    xn = jnp.minimum((1.0 - p) + _CLIP, 1.0)
    one_m_xn = 1.0 - xn
    log_p = jnp.log(jnp.maximum(p, 1e-8))
    log_xn = jnp.log(jnp.maximum(xn, 1e-8))
    pos = t == 1
    una = t == -1
    sq = one_m_xn * one_m_xn
    contrib = jnp.where(
        pos, log_p,
        jnp.where(una, _ALPHA_UNANN * log_xn * sq, log_xn * (sq * sq)))
    partial = jnp.sum(contrib)
    prev = jnp.where(i == 0, 0.0, sum_ref[0, 0])
    sum_ref[0, 0] = prev + partial

    bits = jax.lax.bitcast_convert_type(l, jnp.int32)
    key = jnp.where(bits >= 0, bits, bits ^ _MASK31)
    key = jnp.where(una, key, _INT_MIN)
    rb, c = key.shape
    if cpad > c:
        pad = jnp.full((rb, cpad - c), _INT_MIN, jnp.int32)
        key = jnp.concatenate([key, pad], axis=1)
    keys_ref[...] = key


def _unkey_f(keys):
    lb = jnp.where(keys >= 0, keys, keys ^ _MASK31)
    lv = jax.lax.bitcast_convert_type(lb, jnp.float32)
    p = jax.nn.sigmoid(lv)
    xn = jnp.minimum((1.0 - p) + _CLIP, 1.0)
    f = _ALPHA_UNANN * jnp.log(jnp.maximum(xn, 1e-8)) * (1.0 - xn) ** 2
    return jnp.where(keys == _INT_MIN, 0.0, f)


def _make_hist(B, cpad, rows_w, cc, n_chunks, nbins, shift, pref_shift):
    mesh = plsc.VectorSubcoreMesh(core_axis_name="c", subcore_axis_name="s",
                                  num_cores=2, num_subcores=16)
    cpr = cpad // cc
    nv = cc // 16
    has_prefix = pref_shift is not None

    scratch = [
        pltpu.VMEM((16 * nbins,), jnp.int32),
        pltpu.VMEM((8, cc), jnp.int32),
        pltpu.VMEM((8, cc), jnp.int32),
    ]
    if has_prefix:
        scratch.append(pltpu.VMEM((16,), jnp.int32))
    scratch += [pltpu.SemaphoreType.DMA, pltpu.SemaphoreType.DMA]

    def body(*args):
        if has_prefix:
            (keys_hbm, pref_hbm, out_hbm,
             hist, buf0, buf1, pref_v, sem0, sem1) = args
        else:
            (keys_hbm, out_hbm,
             hist, buf0, buf1, sem0, sem1) = args

        cid = lax.axis_index("c")
        sid = lax.axis_index("s")
        w = sid * 2 + cid
        row0 = w * rows_w

        zeros16 = jnp.zeros((16,), jnp.int32)

        def zb(j, _):
            hist[pl.ds(j * 16, 16)] = zeros16
            return 0
        lax.fori_loop(0, nbins, zb, 0)

        if has_prefix:
            pltpu.sync_copy(pref_hbm, pref_v)
            pref = pref_v[...]

        ones = jnp.ones((16,), jnp.int32)
        lanes = lax.iota(jnp.int32, 16)
        fshift = shift - 4
        fs_v = jnp.full((16,), abs(fshift), jnp.int32)
        fmask = np.int32((nbins - 1) * 16)
        if has_prefix:
            pshift_v = jnp.full((16,), pref_shift, jnp.int32)

        def start(f, buf, sem):
            r = row0 + 8 * (f // cpr)
            c = (f % cpr) * cc
            pltpu.make_async_copy(
                keys_hbm.at[pl.ds(r, 8), pl.ds(c, cc)], buf, sem).start()

        def wait(buf, sem):
            pltpu.make_async_copy(
                keys_hbm.at[pl.ds(row0, 8), pl.ds(0, cc)], buf, sem).wait()

        def process(buf):
            for r in range(8):
                @plsc.parallel_loop(0, nv, 1, unroll=8)
                def pb(j, r=r):
                    kv = buf[r, pl.ds(j * 16, 16)]
                    if fshift >= 0:
                        fld = lax.shift_right_logical(kv, fs_v) & fmask
                    else:
                        fld = lax.shift_left(kv, fs_v) & fmask
                    if has_prefix:
                        okm = lax.shift_right_logical(kv, pshift_v) == pref
                        plsc.addupdate_scatter(hist, [fld + lanes], ones,
                                               mask=okm)
                    else:
                        plsc.addupdate_scatter(hist, [fld + lanes], ones)

        start(0, buf0, sem0)

        def pair(t, _):
            start(2 * t + 1, buf1, sem1)
            wait(buf0, sem0)
            process(buf0)

            @pl.when(t + 1 < n_chunks // 2)
            def _():
                start(2 * t + 2, buf0, sem0)

            wait(buf1, sem1)
            process(buf1)
            return 0

        lax.fori_loop(0, n_chunks // 2, pair, 0)

        pltpu.sync_copy(
            hist, out_hbm.at[pl.ds(w * 16 * nbins, 16 * nbins)])

    return pl.kernel(
        body,
        out_type=jax.ShapeDtypeStruct((_NW * 16 * nbins,), jnp.int32),
        mesh=mesh,
        scratch_types=scratch,
        compiler_params=pltpu.CompilerParams(needs_layout_passes=False),
    )


def _select(h, k_rem):
    c = jnp.cumsum(h[::-1])[::-1]
    jstar = jnp.sum((c >= k_rem).astype(jnp.int32)) - 1
    above = c[jstar] - h[jstar]
    return jstar, k_rem - above


def _radix_threshold(keys, B, cpad, rows_w, cc, n_chunks, k):
    hist1 = _make_hist(B, cpad, rows_w, cc, n_chunks, _NBINS12, 20, None)
    hist2 = _make_hist(B, cpad, rows_w, cc, n_chunks, _NBINS12, 8, 20)
    hist3 = _make_hist(B, cpad, rows_w, cc, n_chunks, _NBINS3, 0, 8)

    h1 = jnp.sum(hist1(keys).reshape(_NW, _NBINS12, 16), axis=(0, 2))
    h1u = h1.reshape(2, _NBINS12 // 2)[::-1].reshape(-1)
    j1u, kr1 = _select(h1u, jnp.int32(k))
    praw1 = j1u ^ (_NBINS12 // 2)
    h2 = jnp.sum(hist2(keys, jnp.full((16,), praw1, jnp.int32))
                 .reshape(_NW, _NBINS12, 16), axis=(0, 2))
    j2, kr2 = _select(h2, kr1)
    praw24 = praw1 * _NBINS12 + j2
    h3 = jnp.sum(hist3(keys, jnp.full((16,), praw24, jnp.int32))
                 .reshape(_NW, _NBINS3, 16), axis=(0, 2))
    j3, m = _select(h3, kr2)

    t_key = (praw24 << 8) | j3
    return t_key, m


def _dpass_body(t_ref, keys_ref, out_ref):
    i = pl.program_id(0)
    t = t_ref[0, 0]
    kv = keys_ref[...]
    f = _unkey_f(kv)
    d = jnp.sum(jnp.where(kv > t, f, 0.0))
    prev = jnp.where(i == 0, 0.0, out_ref[0, 0])
    out_ref[0, 0] = prev + d


@jax.jit
def kernel(logits, targets):
    B, C = logits.shape
    cpad, rows_w, chunk, n_chunks = _plan(B, C)
    RB = 8
    grid = B // RB
    k = _K_PER_ROW * B

    s_all, keys = pl.pallas_call(
        functools.partial(_pass1_body, cpad),
        grid=(grid,),
        in_specs=[
            pl.BlockSpec((RB, C), lambda i: (i, 0)),
            pl.BlockSpec((RB, C), lambda i: (i, 0)),
        ],
        out_specs=[
            pl.BlockSpec((1, 1), lambda i: (0, 0), memory_space=pltpu.SMEM),
            pl.BlockSpec((RB, cpad), lambda i: (i, 0)),
        ],
        out_shape=[
            jax.ShapeDtypeStruct((1, 1), jnp.float32),
            jax.ShapeDtypeStruct((B, cpad), jnp.int32),
        ],
    )(logits, targets)

    t_key, m = _radix_threshold(keys, B, cpad, rows_w, chunk, n_chunks, k)

    d_gt = pl.pallas_call(
        _dpass_body,
        grid=(grid,),
        in_specs=[
            pl.BlockSpec((1, 1), lambda i: (0, 0), memory_space=pltpu.SMEM),
            pl.BlockSpec((RB, cpad), lambda i: (i, 0)),
        ],
        out_specs=pl.BlockSpec((1, 1), lambda i: (0, 0),
                               memory_space=pltpu.SMEM),
        out_shape=jax.ShapeDtypeStruct((1, 1), jnp.float32),
    )(t_key.reshape(1, 1), keys)

    f_t = _unkey_f(t_key.reshape(1))[0]
    d_sum = d_gt[0, 0] + m.astype(jnp.float32) * f_t
    return d_sum - s_all[0, 0]


# speedup vs baseline: 90.9561x; 1.0055x over previous
import functools

import jax
import jax.numpy as jnp
import numpy as np
from jax import lax
from jax.experimental import pallas as pl
from jax.experimental.pallas import tpu as pltpu
from jax.experimental.pallas import tpu_sc as plsc

_CLIP = 0.05
_ALPHA_UNANN = 0.5
_K_PER_ROW = 5
_INT_MIN = np.int32(-2147483648)
_MASK31 = np.int32(2147483647)

_NW = 32
_NBINS12 = 4096
_NBINS3 = 256


def _plan(B, C):
    cpad = ((C + 127) // 128) * 128
    rows_w = B // _NW
    cc = None
    for d in range(3200 - 3200 % 128, 127, -128):
        if cpad % d == 0:
            cc = d
            break
    n_chunks = (rows_w // 8) * (cpad // cc)
    assert n_chunks % 2 == 0 and rows_w % 8 == 0
    return cpad, rows_w, cc, n_chunks


def _pass1_body(cpad, logits_ref, targets_ref, sum_ref, keys_ref):
    i = pl.program_id(0)
    l = logits_ref[...]
    t = targets_ref[...]
    p = jax.nn.sigmoid(l)
    xn = jnp.minimum((1.0 - p) + _CLIP, 1.0)
    one_m_xn = 1.0 - xn
    pos = t == 1
    una = t == -1
    lv = jnp.log(jnp.maximum(jnp.where(pos, p, xn), 1e-8))
    sq = one_m_xn * one_m_xn
    contrib = jnp.where(
        pos, lv,
        jnp.where(una, _ALPHA_UNANN * lv * sq, lv * (sq * sq)))
    partial = jnp.sum(contrib)
    prev = jnp.where(i == 0, 0.0, sum_ref[0, 0])
    sum_ref[0, 0] = prev + partial

    bits = jax.lax.bitcast_convert_type(l, jnp.int32)
    key = jnp.where(bits >= 0, bits, bits ^ _MASK31)
    key = jnp.where(una, key, _INT_MIN)
    rb, c = key.shape
    if cpad > c:
        pad = jnp.full((rb, cpad - c), _INT_MIN, jnp.int32)
        key = jnp.concatenate([key, pad], axis=1)
    keys_ref[...] = key


def _unkey_f(keys):
    lb = jnp.where(keys >= 0, keys, keys ^ _MASK31)
    lv = jax.lax.bitcast_convert_type(lb, jnp.float32)
    p = jax.nn.sigmoid(lv)
    xn = jnp.minimum((1.0 - p) + _CLIP, 1.0)
    f = _ALPHA_UNANN * jnp.log(jnp.maximum(xn, 1e-8)) * (1.0 - xn) ** 2
    return jnp.where(keys == _INT_MIN, 0.0, f)


def _make_hist(B, cpad, rows_w, cc, n_chunks, nbins, shift, pref_shift):
    mesh = plsc.VectorSubcoreMesh(core_axis_name="c", subcore_axis_name="s",
                                  num_cores=2, num_subcores=16)
    cpr = cpad // cc
    nv = cc // 16
    has_prefix = pref_shift is not None

    scratch = [
        pltpu.VMEM((16 * nbins,), jnp.int32),
        pltpu.VMEM((8, cc), jnp.int32),
        pltpu.VMEM((8, cc), jnp.int32),
    ]
    if has_prefix:
        scratch.append(pltpu.VMEM((16,), jnp.int32))
    scratch += [pltpu.SemaphoreType.DMA, pltpu.SemaphoreType.DMA]

    def body(*args):
        if has_prefix:
            (keys_hbm, pref_hbm, out_hbm,
             hist, buf0, buf1, pref_v, sem0, sem1) = args
        else:
            (keys_hbm, out_hbm,
             hist, buf0, buf1, sem0, sem1) = args

        cid = lax.axis_index("c")
        sid = lax.axis_index("s")
        w = sid * 2 + cid
        row0 = w * rows_w

        zeros16 = jnp.zeros((16,), jnp.int32)

        def zb(j, _):
            hist[pl.ds(j * 16, 16)] = zeros16
            return 0
        lax.fori_loop(0, nbins, zb, 0)

        if has_prefix:
            pltpu.sync_copy(pref_hbm, pref_v)
            pref = pref_v[...]

        ones = jnp.ones((16,), jnp.int32)
        lanes = lax.iota(jnp.int32, 16)
        fshift = shift - 4
        fs_v = jnp.full((16,), abs(fshift), jnp.int32)
        fmask = np.int32((nbins - 1) * 16)
        if has_prefix:
            pshift_v = jnp.full((16,), pref_shift, jnp.int32)

        def start(f, buf, sem):
            r = row0 + 8 * (f // cpr)
            c = (f % cpr) * cc
            pltpu.make_async_copy(
                keys_hbm.at[pl.ds(r, 8), pl.ds(c, cc)], buf, sem).start()

        def wait(buf, sem):
            pltpu.make_async_copy(
                keys_hbm.at[pl.ds(row0, 8), pl.ds(0, cc)], buf, sem).wait()

        def process(buf):
            for r in range(8):
                @plsc.parallel_loop(0, nv, 1, unroll=8)
                def pb(j, r=r):
                    kv = buf[r, pl.ds(j * 16, 16)]
                    if fshift >= 0:
                        fld = lax.shift_right_logical(kv, fs_v) & fmask
                    else:
                        fld = lax.shift_left(kv, fs_v) & fmask
                    if has_prefix:
                        okm = lax.shift_right_logical(kv, pshift_v) == pref
                        plsc.addupdate_scatter(hist, [fld + lanes], ones,
                                               mask=okm)
                    else:
                        plsc.addupdate_scatter(hist, [fld + lanes], ones)

        start(0, buf0, sem0)

        def pair(t, _):
            start(2 * t + 1, buf1, sem1)
            wait(buf0, sem0)
            process(buf0)

            @pl.when(t + 1 < n_chunks // 2)
            def _():
                start(2 * t + 2, buf0, sem0)

            wait(buf1, sem1)
            process(buf1)
            return 0

        lax.fori_loop(0, n_chunks // 2, pair, 0)

        pltpu.sync_copy(
            hist, out_hbm.at[pl.ds(w * 16 * nbins, 16 * nbins)])

    return pl.kernel(
        body,
        out_type=jax.ShapeDtypeStruct((_NW * 16 * nbins,), jnp.int32),
        mesh=mesh,
        scratch_types=scratch,
        compiler_params=pltpu.CompilerParams(needs_layout_passes=False),
    )


def _select(h, k_rem):
    c = jnp.cumsum(h[::-1])[::-1]
    jstar = jnp.sum((c >= k_rem).astype(jnp.int32)) - 1
    above = c[jstar] - h[jstar]
    return jstar, k_rem - above


def _radix_threshold(keys, B, cpad, rows_w, cc, n_chunks, k):
    hist1 = _make_hist(B, cpad, rows_w, cc, n_chunks, _NBINS12, 20, None)
    hist2 = _make_hist(B, cpad, rows_w, cc, n_chunks, _NBINS12, 8, 20)
    hist3 = _make_hist(B, cpad, rows_w, cc, n_chunks, _NBINS3, 0, 8)

    h1 = jnp.sum(hist1(keys).reshape(_NW, _NBINS12, 16), axis=(0, 2))
    h1u = h1.reshape(2, _NBINS12 // 2)[::-1].reshape(-1)
    j1u, kr1 = _select(h1u, jnp.int32(k))
    praw1 = j1u ^ (_NBINS12 // 2)
    h2 = jnp.sum(hist2(keys, jnp.full((16,), praw1, jnp.int32))
                 .reshape(_NW, _NBINS12, 16), axis=(0, 2))
    j2, kr2 = _select(h2, kr1)
    praw24 = praw1 * _NBINS12 + j2
    h3 = jnp.sum(hist3(keys, jnp.full((16,), praw24, jnp.int32))
                 .reshape(_NW, _NBINS3, 16), axis=(0, 2))
    j3, m = _select(h3, kr2)

    t_key = (praw24 << 8) | j3
    return t_key, m


def _dpass_body(t_ref, keys_ref, out_ref):
    i = pl.program_id(0)
    t = t_ref[0, 0]
    kv = keys_ref[...]
    f = _unkey_f(kv)
    d = jnp.sum(jnp.where(kv > t, f, 0.0))
    prev = jnp.where(i == 0, 0.0, out_ref[0, 0])
    out_ref[0, 0] = prev + d


@jax.jit
def kernel(logits, targets):
    B, C = logits.shape
    cpad, rows_w, chunk, n_chunks = _plan(B, C)
    RB = 8
    grid = B // RB
    k = _K_PER_ROW * B

    s_all, keys = pl.pallas_call(
        functools.partial(_pass1_body, cpad),
        grid=(grid,),
        in_specs=[
            pl.BlockSpec((RB, C), lambda i: (i, 0)),
            pl.BlockSpec((RB, C), lambda i: (i, 0)),
        ],
        out_specs=[
            pl.BlockSpec((1, 1), lambda i: (0, 0), memory_space=pltpu.SMEM),
            pl.BlockSpec((RB, cpad), lambda i: (i, 0)),
        ],
        out_shape=[
            jax.ShapeDtypeStruct((1, 1), jnp.float32),
            jax.ShapeDtypeStruct((B, cpad), jnp.int32),
        ],
    )(logits, targets)

    t_key, m = _radix_threshold(keys, B, cpad, rows_w, chunk, n_chunks, k)

    d_gt = pl.pallas_call(
        _dpass_body,
        grid=(grid,),
        in_specs=[
            pl.BlockSpec((1, 1), lambda i: (0, 0), memory_space=pltpu.SMEM),
            pl.BlockSpec((RB, cpad), lambda i: (i, 0)),
        ],
        out_specs=pl.BlockSpec((1, 1), lambda i: (0, 0),
                               memory_space=pltpu.SMEM),
        out_shape=jax.ShapeDtypeStruct((1, 1), jnp.float32),
    )(t_key.reshape(1, 1), keys)

    f_t = _unkey_f(t_key.reshape(1))[0]
    d_sum = d_gt[0, 0] + m.astype(jnp.float32) * f_t
    return d_sum - s_all[0, 0]
